# baseline (speedup 1.0000x reference)
import jax
import jax.numpy as jnp
from jax import lax
from jax.experimental import pallas as pl
from jax.experimental.pallas import tpu as pltpu
from jax.experimental.pallas import tpu_sc as plsc

N = 10000
E = 320000
D_IN = 128
H = 32
D_OUT = 64

NP = 10240
NTILES = 16
SEG = NP // NTILES
CH = 128
EPT = E // NTILES
NB = 4
NCH = 160
EPTP = NCH * CH
HALF = 16
SEGR = SEG // 16


def _rsqrt_approx(d):
    i = lax.bitcast_convert_type(d, jnp.int32)
    i = jnp.int32(0x5F3759DF) - lax.shift_right_logical(i, 1)
    y = lax.bitcast_convert_type(i, jnp.float32)
    for _ in range(3):
        y = y * (jnp.float32(1.5) - jnp.float32(0.5) * d * y * y)
    return y


def _mm1_body(x_ref, w_ref, o_ref):
    x = x_ref[...]
    w = w_ref[...]
    o_ref[0] = jnp.dot(x, w[:, :HALF], preferred_element_type=jnp.float32)
    o_ref[1] = jnp.dot(x, w[:, HALF:], preferred_element_type=jnp.float32)


def _mm1(x, W1):
    bm = 1024
    return pl.pallas_call(
        _mm1_body,
        grid=(NP // bm,),
        in_specs=[
            pl.BlockSpec((bm, D_IN), lambda i: (i, 0)),
            pl.BlockSpec((D_IN, H), lambda i: (0, 0)),
        ],
        out_specs=pl.BlockSpec((2, bm, HALF), lambda i: (0, i, 0)),
        out_shape=jax.ShapeDtypeStruct((2, NP, HALF), jnp.float32),
    )(x, W1)


def _msg_body(y1_hbm, src_hbm, dst_hbm, b1_hbm,
              out_hbm, deg_hbm, z_hbm,
              src_v, dst_v, buf_v, zbuf_v, ebuf_v,
              ones_v, nsrc_v, ndst_v, dv_v, b1_v,
              agg_s, hs_s, hd_s,
              sem, gsem, ssem):
    c = lax.axis_index("c")
    t = lax.axis_index("s")
    base = t * SEG

    pltpu.sync_copy(src_hbm.at[t], src_v.at[pl.ds(0, EPT)])
    pltpu.sync_copy(dst_hbm.at[t], dst_v.at[pl.ds(0, EPT)])
    pltpu.sync_copy(b1_hbm.at[pl.ds(c * HALF, HALF)], b1_v)
    lane = lax.iota(jnp.int32, 16)
    for k in range((EPTP - EPT) // 16):
        pv = jnp.int32(N + (k * 16) % 240) + lane
        src_v[pl.ds(EPT + k * 16, 16)] = pv
        dst_v[pl.ds(EPT + k * 16, 16)] = pv

    for k in range(CH // 16):
        ones_v[pl.ds(k * 16, 16)] = jnp.ones((16,), jnp.float32)

    def _z1d(k, carry):
        dv_v[pl.ds(k * 16, 16)] = jnp.zeros((16,), jnp.float32)
        return carry
    lax.fori_loop(0, SEG // 16, _z1d, 0)

    pltpu.sync_copy(dv_v, hs_s.at[pl.ds(base, SEG)])
    pltpu.sync_copy(dv_v, hd_s.at[pl.ds(base, SEG)])
    plsc.subcore_barrier()

    def _hist(j, carry):
        pltpu.async_copy(ones_v, hs_s.at[src_v.at[pl.ds(j * CH, CH)]],
                         sem, add=True)
        pltpu.async_copy(ones_v, hd_s.at[dst_v.at[pl.ds(j * CH, CH)]],
                         sem, add=True)
        return carry
    lax.fori_loop(0, NCH, _hist, 0)

    def _zrow(r, carry):
        zbuf_v[r, :] = jnp.zeros((16,), jnp.float32)
        return carry
    lax.fori_loop(0, SEG, _zrow, 0)
    pltpu.sync_copy(y1_hbm.at[pl.ds(c * NP + base, SEG)], buf_v)

    def _hdrain(j, carry):
        pltpu.make_async_copy(ones_v, hs_s.at[src_v.at[pl.ds(0, CH)]],
                              sem).wait()
        pltpu.make_async_copy(ones_v, hd_s.at[dst_v.at[pl.ds(0, CH)]],
                              sem).wait()
        return carry
    lax.fori_loop(0, NCH, _hdrain, 0)
    plsc.subcore_barrier()

    def _norms(out_ref):
        def body(k, carry):
            d = jnp.maximum(dv_v[pl.ds(k * 16, 16)], jnp.float32(1.0))
            out_ref[pl.ds(k * 16, 16)] = _rsqrt_approx(d)
            return carry
        lax.fori_loop(0, SEG // 16, body, 0)

    pltpu.sync_copy(hs_s.at[pl.ds(base, SEG)], dv_v)
    _norms(nsrc_v)
    pltpu.sync_copy(hd_s.at[pl.ds(base, SEG)], dv_v)

    @pl.when(c == 0)
    def _():
        pltpu.sync_copy(dv_v, deg_hbm.at[pl.ds(base, SEG)])
    _norms(ndst_v)

    def _scale(k, carry):
        nv = nsrc_v[pl.ds(k * 16, 16)]
        for l in range(16):
            r = k * 16 + l
            buf_v[r, :] = buf_v[r, :] * nv[l]
        return carry
    lax.fori_loop(0, SEG // 16, _scale, 0)
    pltpu.sync_copy(buf_v, z_hbm.at[c].at[pl.ds(base, SEG)])
    pltpu.sync_copy(zbuf_v, agg_s.at[pl.ds(base, SEG)])
    plsc.subcore_barrier()

    def _g_issue(j, b):
        pltpu.async_copy(z_hbm.at[c].at[src_v.at[pl.ds(j * CH, CH)]],
                         ebuf_v.at[b], gsem.at[b])

    def _g_wait(b):
        pltpu.make_async_copy(z_hbm.at[c].at[src_v.at[pl.ds(0, CH)]],
                              ebuf_v.at[b], gsem.at[b]).wait()

    def _s_issue(j, b):
        pltpu.async_copy(ebuf_v.at[b], agg_s.at[dst_v.at[pl.ds(j * CH, CH)]],
                         ssem.at[b], add=True)

    def _s_wait(b):
        pltpu.make_async_copy(ebuf_v.at[b], agg_s.at[dst_v.at[pl.ds(0, CH)]],
                              ssem.at[b]).wait()

    def _edges():
        for b in range(NB):
            _g_issue(b, b)
        for b in range(NB - 1):
            _g_wait(b)
            _s_issue(b, b)

        def _body(o, carry):
            for b in range(NB):
                j = NB + o * NB + b
                _s_wait(b)
                _g_issue(j, b)
                b1 = (b + NB - 1) % NB
                _g_wait(b1)
                _s_issue(j - 1, b1)
            return carry
        lax.fori_loop(0, (NCH - NB) // NB, _body, 0)

        _g_wait((NCH - 1) % NB)
        _s_issue(NCH - 1, (NCH - 1) % NB)
        for b in range(NB):
            _s_wait(b)

    _edges()
    plsc.subcore_barrier()

    pltpu.sync_copy(agg_s.at[pl.ds(base, SEG)], buf_v)
    b1row = b1_v[...]

    def _mid(k, carry):
        nvd = ndst_v[pl.ds(k * 16, 16)]
        nvs = nsrc_v[pl.ds(k * 16, 16)]
        for l in range(16):
            r = k * 16 + l
            h = jnp.maximum(buf_v[r, :] * nvd[l] + b1row, jnp.float32(0.0))
            buf_v[r, :] = h * nvs[l]
        return carry
    lax.fori_loop(0, SEG // 16, _mid, 0)
    pltpu.sync_copy(buf_v, z_hbm.at[c].at[pl.ds(base, SEG)])
    pltpu.sync_copy(zbuf_v, agg_s.at[pl.ds(base, SEG)])
    plsc.subcore_barrier()

    _edges()
    plsc.subcore_barrier()

    pltpu.sync_copy(agg_s.at[pl.ds(base, SEG)], buf_v)
    pltpu.sync_copy(buf_v, out_hbm.at[pl.ds(c * NP + base, SEG)])


_msgpass = pl.kernel(
    _msg_body,
    out_type=(
        jax.ShapeDtypeStruct((2 * NP, HALF), jnp.float32),
        jax.ShapeDtypeStruct((NP,), jnp.float32),
        jax.ShapeDtypeStruct((2, NP, HALF), jnp.float32),
    ),
    mesh=plsc.VectorSubcoreMesh(core_axis_name="c", subcore_axis_name="s"),
    compiler_params=pltpu.CompilerParams(use_tc_tiling_on_sc=False),
    scratch_types=[
        pltpu.VMEM((EPTP,), jnp.int32),
        pltpu.VMEM((EPTP,), jnp.int32),
        pltpu.VMEM((SEG, HALF), jnp.float32),
        pltpu.VMEM((SEG, HALF), jnp.float32),
        pltpu.VMEM((NB, CH, HALF), jnp.float32),
        pltpu.VMEM((CH,), jnp.float32),
        pltpu.VMEM((SEG,), jnp.float32),
        pltpu.VMEM((SEG,), jnp.float32),
        pltpu.VMEM((SEG,), jnp.float32),
        pltpu.VMEM((HALF,), jnp.float32),
        pltpu.VMEM_SHARED((NP, HALF), jnp.float32),
        pltpu.VMEM_SHARED((NP,), jnp.float32),
        pltpu.VMEM_SHARED((NP,), jnp.float32),
        pltpu.SemaphoreType.DMA,
        pltpu.SemaphoreType.DMA((NB,)),
        pltpu.SemaphoreType.DMA((NB,)),
    ],
)


def _fin_body(a_ref, d_ref, w_ref, b_ref, o_ref):
    a2 = a_ref[...]
    a = jnp.concatenate([a2[0], a2[1]], axis=1)
    nd = lax.rsqrt(jnp.maximum(d_ref[...], jnp.float32(1.0)))
    o = jnp.dot(a * nd, w_ref[...], preferred_element_type=jnp.float32)
    o = o + b_ref[...]
    m = jnp.max(o, axis=1, keepdims=True)
    e = o - m
    lse = jnp.log(jnp.sum(jnp.exp(e), axis=1, keepdims=True))
    o_ref[...] = e - lse


def _final(agg, deg, W2, b2):
    bm = 1024
    return pl.pallas_call(
        _fin_body,
        grid=(NP // bm,),
        in_specs=[
            pl.BlockSpec((2, bm, HALF), lambda i: (0, i, 0)),
            pl.BlockSpec((bm, 1), lambda i: (i, 0)),
            pl.BlockSpec((H, D_OUT), lambda i: (0, 0)),
            pl.BlockSpec((1, D_OUT), lambda i: (0, 0)),
        ],
        out_specs=pl.BlockSpec((bm, D_OUT), lambda i: (i, 0)),
        out_shape=jax.ShapeDtypeStruct((N, D_OUT), jnp.float32),
    )(agg, deg, W2, b2)


@jax.jit
def kernel(x, edge_index, W1, b1, W2, b2):
    srcr = edge_index[0].reshape(NTILES, EPT)
    dstr = edge_index[1].reshape(NTILES, EPT)
    y1 = _mm1(x, W1).reshape(2 * NP, HALF)
    agg, deg, _ = _msgpass(y1, srcr, dstr, b1)
    return _final(agg.reshape(2, NP, HALF), deg.reshape(NP, 1),
                  W2, b2.reshape(1, D_OUT))


# speedup vs baseline: 1.4213x; 1.4213x over previous
import jax
import jax.numpy as jnp
from jax import lax
from jax.experimental import pallas as pl
from jax.experimental.pallas import tpu as pltpu
from jax.experimental.pallas import tpu_sc as plsc

N = 10000
E = 320000
D_IN = 128
H = 32
D_OUT = 64

NP = 10240
NTILES = 16
SEG = NP // NTILES
CH = 128
EPT = E // NTILES
NB = 8
NCH = 160
EPTP = NCH * CH
HALF = 16
SEGR = SEG // 16


def _rsqrt_approx(d):
    i = lax.bitcast_convert_type(d, jnp.int32)
    i = jnp.int32(0x5F3759DF) - lax.shift_right_logical(i, 1)
    y = lax.bitcast_convert_type(i, jnp.float32)
    for _ in range(3):
        y = y * (jnp.float32(1.5) - jnp.float32(0.5) * d * y * y)
    return y


def _mm1_body(x_ref, w_ref, o_ref):
    x = x_ref[...]
    w = w_ref[...]
    o_ref[0] = jnp.dot(x, w[:, :HALF], preferred_element_type=jnp.float32)
    o_ref[1] = jnp.dot(x, w[:, HALF:], preferred_element_type=jnp.float32)


def _mm1(x, W1):
    bm = 1024
    return pl.pallas_call(
        _mm1_body,
        grid=(NP // bm,),
        in_specs=[
            pl.BlockSpec((bm, D_IN), lambda i: (i, 0)),
            pl.BlockSpec((D_IN, H), lambda i: (0, 0)),
        ],
        out_specs=pl.BlockSpec((2, bm, HALF), lambda i: (0, i, 0)),
        out_shape=jax.ShapeDtypeStruct((2, NP, HALF), jnp.float32),
    )(x, W1)


def _msg_body(y1_hbm, src_hbm, dst_hbm, b1_hbm,
              out_hbm, deg_hbm,
              src_v, dst_v, buf_v, zbuf_v, ebuf_v,
              ones_v, nsrc_v, ndst_v, dv_v, b1_v,
              z_s, agg_s, hs_s, hd_s,
              sem, gsem, ssem):
    c = lax.axis_index("c")
    t = lax.axis_index("s")
    base = t * SEG

    pltpu.sync_copy(src_hbm.at[t], src_v.at[pl.ds(0, EPT)])
    pltpu.sync_copy(dst_hbm.at[t], dst_v.at[pl.ds(0, EPT)])
    pltpu.sync_copy(b1_hbm.at[pl.ds(c * HALF, HALF)], b1_v)
    lane = lax.iota(jnp.int32, 16)
    for k in range((EPTP - EPT) // 16):
        pv = jnp.int32(N + (k * 16) % 240) + lane
        src_v[pl.ds(EPT + k * 16, 16)] = pv
        dst_v[pl.ds(EPT + k * 16, 16)] = pv

    for k in range(CH // 16):
        ones_v[pl.ds(k * 16, 16)] = jnp.ones((16,), jnp.float32)

    def _z1d(k, carry):
        dv_v[pl.ds(k * 16, 16)] = jnp.zeros((16,), jnp.float32)
        return carry
    lax.fori_loop(0, SEG // 16, _z1d, 0)

    pltpu.sync_copy(dv_v, hs_s.at[pl.ds(base, SEG)])
    pltpu.sync_copy(dv_v, hd_s.at[pl.ds(base, SEG)])
    plsc.subcore_barrier()

    def _hist(j, carry):
        pltpu.async_copy(ones_v, hs_s.at[src_v.at[pl.ds(j * CH, CH)]],
                         sem, add=True)
        pltpu.async_copy(ones_v, hd_s.at[dst_v.at[pl.ds(j * CH, CH)]],
                         sem, add=True)
        return carry
    lax.fori_loop(0, NCH, _hist, 0)

    def _zrow(r, carry):
        zbuf_v[r, :] = jnp.zeros((16,), jnp.float32)
        return carry
    lax.fori_loop(0, SEG, _zrow, 0)
    pltpu.sync_copy(y1_hbm.at[pl.ds(c * NP + base, SEG)], buf_v)

    def _hdrain(j, carry):
        pltpu.make_async_copy(ones_v, hs_s.at[src_v.at[pl.ds(0, CH)]],
                              sem).wait()
        pltpu.make_async_copy(ones_v, hd_s.at[dst_v.at[pl.ds(0, CH)]],
                              sem).wait()
        return carry
    lax.fori_loop(0, NCH, _hdrain, 0)
    plsc.subcore_barrier()

    def _norms(out_ref):
        def body(k, carry):
            d = jnp.maximum(dv_v[pl.ds(k * 16, 16)], jnp.float32(1.0))
            out_ref[pl.ds(k * 16, 16)] = _rsqrt_approx(d)
            return carry
        lax.fori_loop(0, SEG // 16, body, 0)

    pltpu.sync_copy(hs_s.at[pl.ds(base, SEG)], dv_v)
    _norms(nsrc_v)
    pltpu.sync_copy(hd_s.at[pl.ds(base, SEG)], dv_v)

    @pl.when(c == 0)
    def _():
        pltpu.sync_copy(dv_v, deg_hbm.at[pl.ds(base, SEG)])
    _norms(ndst_v)

    def _scale(k, carry):
        nv = nsrc_v[pl.ds(k * 16, 16)]
        for l in range(16):
            r = k * 16 + l
            buf_v[r, :] = buf_v[r, :] * nv[l]
        return carry
    lax.fori_loop(0, SEG // 16, _scale, 0)
    pltpu.sync_copy(buf_v, z_s.at[pl.ds(base, SEG)])
    pltpu.sync_copy(zbuf_v, agg_s.at[pl.ds(base, SEG)])
    plsc.subcore_barrier()

    def _g_issue(j, b):
        pltpu.async_copy(z_s.at[src_v.at[pl.ds(j * CH, CH)]], ebuf_v.at[b],
                         gsem.at[b])

    def _g_wait(b):
        pltpu.make_async_copy(z_s.at[src_v.at[pl.ds(0, CH)]], ebuf_v.at[b],
                              gsem.at[b]).wait()

    def _s_issue(j, b):
        pltpu.async_copy(ebuf_v.at[b], agg_s.at[dst_v.at[pl.ds(j * CH, CH)]],
                         ssem.at[b], add=True)

    def _s_wait(b):
        pltpu.make_async_copy(ebuf_v.at[b], agg_s.at[dst_v.at[pl.ds(0, CH)]],
                              ssem.at[b]).wait()

    def _edges():
        for b in range(NB):
            _g_issue(b, b)
        for b in range(NB - 1):
            _g_wait(b)
            _s_issue(b, b)

        def _body(o, carry):
            for b in range(NB):
                j = NB + o * NB + b
                _s_wait(b)
                _g_issue(j, b)
                b1 = (b + NB - 1) % NB
                _g_wait(b1)
                _s_issue(j - 1, b1)
            return carry
        lax.fori_loop(0, (NCH - NB) // NB, _body, 0)

        _g_wait((NCH - 1) % NB)
        _s_issue(NCH - 1, (NCH - 1) % NB)
        for b in range(NB):
            _s_wait(b)

    _edges()
    plsc.subcore_barrier()

    pltpu.sync_copy(agg_s.at[pl.ds(base, SEG)], buf_v)
    b1row = b1_v[...]

    def _mid(k, carry):
        nvd = ndst_v[pl.ds(k * 16, 16)]
        nvs = nsrc_v[pl.ds(k * 16, 16)]
        for l in range(16):
            r = k * 16 + l
            h = jnp.maximum(buf_v[r, :] * nvd[l] + b1row, jnp.float32(0.0))
            buf_v[r, :] = h * nvs[l]
        return carry
    lax.fori_loop(0, SEG // 16, _mid, 0)
    pltpu.sync_copy(buf_v, z_s.at[pl.ds(base, SEG)])
    pltpu.sync_copy(zbuf_v, agg_s.at[pl.ds(base, SEG)])
    plsc.subcore_barrier()

    _edges()
    plsc.subcore_barrier()

    pltpu.sync_copy(agg_s.at[pl.ds(base, SEG)], buf_v)
    pltpu.sync_copy(buf_v, out_hbm.at[pl.ds(c * NP + base, SEG)])


_msgpass = pl.kernel(
    _msg_body,
    out_type=(
        jax.ShapeDtypeStruct((2 * NP, HALF), jnp.float32),
        jax.ShapeDtypeStruct((NP,), jnp.float32),
    ),
    mesh=plsc.VectorSubcoreMesh(core_axis_name="c", subcore_axis_name="s"),
    compiler_params=pltpu.CompilerParams(use_tc_tiling_on_sc=False),
    scratch_types=[
        pltpu.VMEM((EPTP,), jnp.int32),
        pltpu.VMEM((EPTP,), jnp.int32),
        pltpu.VMEM((SEG, HALF), jnp.float32),
        pltpu.VMEM((SEG, HALF), jnp.float32),
        pltpu.VMEM((NB, CH, HALF), jnp.float32),
        pltpu.VMEM((CH,), jnp.float32),
        pltpu.VMEM((SEG,), jnp.float32),
        pltpu.VMEM((SEG,), jnp.float32),
        pltpu.VMEM((SEG,), jnp.float32),
        pltpu.VMEM((HALF,), jnp.float32),
        pltpu.VMEM_SHARED((NP, HALF), jnp.float32),
        pltpu.VMEM_SHARED((NP, HALF), jnp.float32),
        pltpu.VMEM_SHARED((NP,), jnp.float32),
        pltpu.VMEM_SHARED((NP,), jnp.float32),
        pltpu.SemaphoreType.DMA,
        pltpu.SemaphoreType.DMA((NB,)),
        pltpu.SemaphoreType.DMA((NB,)),
    ],
)


def _fin_body(a_ref, d_ref, w_ref, b_ref, o_ref):
    a2 = a_ref[...]
    a = jnp.concatenate([a2[0], a2[1]], axis=1)
    nd = lax.rsqrt(jnp.maximum(d_ref[...], jnp.float32(1.0)))
    o = jnp.dot(a * nd, w_ref[...], preferred_element_type=jnp.float32)
    o = o + b_ref[...]
    m = jnp.max(o, axis=1, keepdims=True)
    e = o - m
    lse = jnp.log(jnp.sum(jnp.exp(e), axis=1, keepdims=True))
    o_ref[...] = e - lse


def _final(agg, deg, W2, b2):
    bm = 1024
    return pl.pallas_call(
        _fin_body,
        grid=(NP // bm,),
        in_specs=[
            pl.BlockSpec((2, bm, HALF), lambda i: (0, i, 0)),
            pl.BlockSpec((bm, 1), lambda i: (i, 0)),
            pl.BlockSpec((H, D_OUT), lambda i: (0, 0)),
            pl.BlockSpec((1, D_OUT), lambda i: (0, 0)),
        ],
        out_specs=pl.BlockSpec((bm, D_OUT), lambda i: (i, 0)),
        out_shape=jax.ShapeDtypeStruct((N, D_OUT), jnp.float32),
    )(agg, deg, W2, b2)


@jax.jit
def kernel(x, edge_index, W1, b1, W2, b2):
    srcr = edge_index[0].reshape(NTILES, EPT)
    dstr = edge_index[1].reshape(NTILES, EPT)
    y1 = _mm1(x, W1).reshape(2 * NP, HALF)
    agg, deg = _msgpass(y1, srcr, dstr, b1)
    return _final(agg.reshape(2, NP, HALF), deg.reshape(NP, 1),
                  W2, b2.reshape(1, D_OUT))


# speedup vs baseline: 1.4858x; 1.0453x over previous
import jax
import jax.numpy as jnp
from jax import lax
from jax.experimental import pallas as pl
from jax.experimental.pallas import tpu as pltpu
from jax.experimental.pallas import tpu_sc as plsc

N = 10000
E = 320000
D_IN = 128
H = 32
D_OUT = 64

NP = 10240
NTILES = 16
SEG = NP // NTILES
CH = 256
EPT = E // NTILES
NB = 4
NCH = 80
EPTP = NCH * CH
HALF = 16
SEGR = SEG // 16


def _rsqrt_approx(d):
    i = lax.bitcast_convert_type(d, jnp.int32)
    i = jnp.int32(0x5F3759DF) - lax.shift_right_logical(i, 1)
    y = lax.bitcast_convert_type(i, jnp.float32)
    for _ in range(3):
        y = y * (jnp.float32(1.5) - jnp.float32(0.5) * d * y * y)
    return y


def _mm1_body(x_ref, w_ref, o_ref):
    x = x_ref[...]
    w = w_ref[...]
    o_ref[0] = jnp.dot(x, w[:, :HALF], preferred_element_type=jnp.float32)
    o_ref[1] = jnp.dot(x, w[:, HALF:], preferred_element_type=jnp.float32)


def _mm1(x, W1):
    bm = 1024
    return pl.pallas_call(
        _mm1_body,
        grid=(NP // bm,),
        in_specs=[
            pl.BlockSpec((bm, D_IN), lambda i: (i, 0)),
            pl.BlockSpec((D_IN, H), lambda i: (0, 0)),
        ],
        out_specs=pl.BlockSpec((2, bm, HALF), lambda i: (0, i, 0)),
        out_shape=jax.ShapeDtypeStruct((2, NP, HALF), jnp.float32),
    )(x, W1)


def _msg_body(y1_hbm, src_hbm, dst_hbm, b1_hbm,
              out_hbm, deg_hbm,
              src_v, dst_v, buf_v, zbuf_v, ebuf_v,
              ones_v, nsrc_v, ndst_v, dv_v, b1_v,
              z_s, agg_s, hs_s, hd_s,
              sem, gsem, ssem):
    c = lax.axis_index("c")
    t = lax.axis_index("s")
    base = t * SEG

    pltpu.sync_copy(src_hbm.at[t], src_v.at[pl.ds(0, EPT)])
    pltpu.sync_copy(dst_hbm.at[t], dst_v.at[pl.ds(0, EPT)])
    pltpu.sync_copy(b1_hbm.at[pl.ds(c * HALF, HALF)], b1_v)
    lane = lax.iota(jnp.int32, 16)
    for k in range((EPTP - EPT) // 16):
        pv = jnp.int32(N + (k * 16) % 240) + lane
        src_v[pl.ds(EPT + k * 16, 16)] = pv
        dst_v[pl.ds(EPT + k * 16, 16)] = pv

    for k in range(CH // 16):
        ones_v[pl.ds(k * 16, 16)] = jnp.ones((16,), jnp.float32)

    def _z1d(k, carry):
        dv_v[pl.ds(k * 16, 16)] = jnp.zeros((16,), jnp.float32)
        return carry
    lax.fori_loop(0, SEG // 16, _z1d, 0)

    pltpu.sync_copy(dv_v, hs_s.at[pl.ds(base, SEG)])
    pltpu.sync_copy(dv_v, hd_s.at[pl.ds(base, SEG)])
    plsc.subcore_barrier()

    def _hist(j, carry):
        pltpu.async_copy(ones_v, hs_s.at[src_v.at[pl.ds(j * CH, CH)]],
                         sem, add=True)
        pltpu.async_copy(ones_v, hd_s.at[dst_v.at[pl.ds(j * CH, CH)]],
                         sem, add=True)
        return carry
    lax.fori_loop(0, NCH, _hist, 0)

    def _zrow(r, carry):
        zbuf_v[r, :] = jnp.zeros((16,), jnp.float32)
        return carry
    lax.fori_loop(0, SEG, _zrow, 0)
    pltpu.sync_copy(y1_hbm.at[pl.ds(c * NP + base, SEG)], buf_v)

    def _hdrain(j, carry):
        pltpu.make_async_copy(ones_v, hs_s.at[src_v.at[pl.ds(0, CH)]],
                              sem).wait()
        pltpu.make_async_copy(ones_v, hd_s.at[dst_v.at[pl.ds(0, CH)]],
                              sem).wait()
        return carry
    lax.fori_loop(0, NCH, _hdrain, 0)
    plsc.subcore_barrier()

    def _norms(out_ref):
        def body(k, carry):
            d = jnp.maximum(dv_v[pl.ds(k * 16, 16)], jnp.float32(1.0))
            out_ref[pl.ds(k * 16, 16)] = _rsqrt_approx(d)
            return carry
        lax.fori_loop(0, SEG // 16, body, 0)

    pltpu.sync_copy(hs_s.at[pl.ds(base, SEG)], dv_v)
    _norms(nsrc_v)
    pltpu.sync_copy(hd_s.at[pl.ds(base, SEG)], dv_v)

    @pl.when(c == 0)
    def _():
        pltpu.sync_copy(dv_v, deg_hbm.at[pl.ds(base, SEG)])
    _norms(ndst_v)

    def _scale(k, carry):
        nv = nsrc_v[pl.ds(k * 16, 16)]
        for l in range(16):
            r = k * 16 + l
            buf_v[r, :] = buf_v[r, :] * nv[l]
        return carry
    lax.fori_loop(0, SEG // 16, _scale, 0)
    pltpu.sync_copy(buf_v, z_s.at[pl.ds(base, SEG)])
    pltpu.sync_copy(zbuf_v, agg_s.at[pl.ds(base, SEG)])
    plsc.subcore_barrier()

    def _g_issue(j, b):
        pltpu.async_copy(z_s.at[src_v.at[pl.ds(j * CH, CH)]], ebuf_v.at[b],
                         gsem.at[b])

    def _g_wait(b):
        pltpu.make_async_copy(z_s.at[src_v.at[pl.ds(0, CH)]], ebuf_v.at[b],
                              gsem.at[b]).wait()

    def _s_issue(j, b):
        pltpu.async_copy(ebuf_v.at[b], agg_s.at[dst_v.at[pl.ds(j * CH, CH)]],
                         ssem.at[b], add=True)

    def _s_wait(b):
        pltpu.make_async_copy(ebuf_v.at[b], agg_s.at[dst_v.at[pl.ds(0, CH)]],
                              ssem.at[b]).wait()

    def _edges():
        for b in range(NB):
            _g_issue(b, b)
        for b in range(NB - 1):
            _g_wait(b)
            _s_issue(b, b)

        def _body(o, carry):
            for b in range(NB):
                j = NB + o * NB + b
                _s_wait(b)
                _g_issue(j, b)
                b1 = (b + NB - 1) % NB
                _g_wait(b1)
                _s_issue(j - 1, b1)
            return carry
        lax.fori_loop(0, (NCH - NB) // NB, _body, 0)

        _g_wait((NCH - 1) % NB)
        _s_issue(NCH - 1, (NCH - 1) % NB)
        for b in range(NB):
            _s_wait(b)

    _edges()
    plsc.subcore_barrier()

    pltpu.sync_copy(agg_s.at[pl.ds(base, SEG)], buf_v)
    b1row = b1_v[...]

    def _mid(k, carry):
        nvd = ndst_v[pl.ds(k * 16, 16)]
        nvs = nsrc_v[pl.ds(k * 16, 16)]
        for l in range(16):
            r = k * 16 + l
            h = jnp.maximum(buf_v[r, :] * nvd[l] + b1row, jnp.float32(0.0))
            buf_v[r, :] = h * nvs[l]
        return carry
    lax.fori_loop(0, SEG // 16, _mid, 0)
    pltpu.sync_copy(buf_v, z_s.at[pl.ds(base, SEG)])
    pltpu.sync_copy(zbuf_v, agg_s.at[pl.ds(base, SEG)])
    plsc.subcore_barrier()

    _edges()
    plsc.subcore_barrier()

    pltpu.sync_copy(agg_s.at[pl.ds(base, SEG)], buf_v)
    pltpu.sync_copy(buf_v, out_hbm.at[pl.ds(c * NP + base, SEG)])


_msgpass = pl.kernel(
    _msg_body,
    out_type=(
        jax.ShapeDtypeStruct((2 * NP, HALF), jnp.float32),
        jax.ShapeDtypeStruct((NP,), jnp.float32),
    ),
    mesh=plsc.VectorSubcoreMesh(core_axis_name="c", subcore_axis_name="s"),
    compiler_params=pltpu.CompilerParams(use_tc_tiling_on_sc=False),
    scratch_types=[
        pltpu.VMEM((EPTP,), jnp.int32),
        pltpu.VMEM((EPTP,), jnp.int32),
        pltpu.VMEM((SEG, HALF), jnp.float32),
        pltpu.VMEM((SEG, HALF), jnp.float32),
        pltpu.VMEM((NB, CH, HALF), jnp.float32),
        pltpu.VMEM((CH,), jnp.float32),
        pltpu.VMEM((SEG,), jnp.float32),
        pltpu.VMEM((SEG,), jnp.float32),
        pltpu.VMEM((SEG,), jnp.float32),
        pltpu.VMEM((HALF,), jnp.float32),
        pltpu.VMEM_SHARED((NP, HALF), jnp.float32),
        pltpu.VMEM_SHARED((NP, HALF), jnp.float32),
        pltpu.VMEM_SHARED((NP,), jnp.float32),
        pltpu.VMEM_SHARED((NP,), jnp.float32),
        pltpu.SemaphoreType.DMA,
        pltpu.SemaphoreType.DMA((NB,)),
        pltpu.SemaphoreType.DMA((NB,)),
    ],
)


def _fin_body(a_ref, d_ref, w_ref, b_ref, o_ref):
    a2 = a_ref[...]
    a = jnp.concatenate([a2[0], a2[1]], axis=1)
    nd = lax.rsqrt(jnp.maximum(d_ref[...], jnp.float32(1.0)))
    o = jnp.dot(a * nd, w_ref[...], preferred_element_type=jnp.float32)
    o = o + b_ref[...]
    m = jnp.max(o, axis=1, keepdims=True)
    e = o - m
    lse = jnp.log(jnp.sum(jnp.exp(e), axis=1, keepdims=True))
    o_ref[...] = e - lse


def _final(agg, deg, W2, b2):
    bm = 1024
    return pl.pallas_call(
        _fin_body,
        grid=(NP // bm,),
        in_specs=[
            pl.BlockSpec((2, bm, HALF), lambda i: (0, i, 0)),
            pl.BlockSpec((bm, 1), lambda i: (i, 0)),
            pl.BlockSpec((H, D_OUT), lambda i: (0, 0)),
            pl.BlockSpec((1, D_OUT), lambda i: (0, 0)),
        ],
        out_specs=pl.BlockSpec((bm, D_OUT), lambda i: (i, 0)),
        out_shape=jax.ShapeDtypeStruct((N, D_OUT), jnp.float32),
    )(agg, deg, W2, b2)


@jax.jit
def kernel(x, edge_index, W1, b1, W2, b2):
    srcr = edge_index[0].reshape(NTILES, EPT)
    dstr = edge_index[1].reshape(NTILES, EPT)
    y1 = _mm1(x, W1).reshape(2 * NP, HALF)
    agg, deg = _msgpass(y1, srcr, dstr, b1)
    return _final(agg.reshape(2, NP, HALF), deg.reshape(NP, 1),
                  W2, b2.reshape(1, D_OUT))


# speedup vs baseline: 1.5059x; 1.0136x over previous
import jax
import jax.numpy as jnp
from jax import lax
from jax.experimental import pallas as pl
from jax.experimental.pallas import tpu as pltpu
from jax.experimental.pallas import tpu_sc as plsc

N = 10000
E = 320000
D_IN = 128
H = 32
D_OUT = 64

NP = 10240
NTILES = 16
SEG = NP // NTILES
CH = 512
EPT = E // NTILES
NB = 4
NCH = 40
EPTP = NCH * CH
HALF = 16
SEGR = SEG // 16


def _rsqrt_approx(d):
    i = lax.bitcast_convert_type(d, jnp.int32)
    i = jnp.int32(0x5F3759DF) - lax.shift_right_logical(i, 1)
    y = lax.bitcast_convert_type(i, jnp.float32)
    for _ in range(3):
        y = y * (jnp.float32(1.5) - jnp.float32(0.5) * d * y * y)
    return y


def _mm1_body(x_ref, w_ref, o_ref):
    x = x_ref[...]
    w = w_ref[...]
    o_ref[0] = jnp.dot(x, w[:, :HALF], preferred_element_type=jnp.float32)
    o_ref[1] = jnp.dot(x, w[:, HALF:], preferred_element_type=jnp.float32)


def _mm1(x, W1):
    bm = 1024
    return pl.pallas_call(
        _mm1_body,
        grid=(NP // bm,),
        in_specs=[
            pl.BlockSpec((bm, D_IN), lambda i: (i, 0)),
            pl.BlockSpec((D_IN, H), lambda i: (0, 0)),
        ],
        out_specs=pl.BlockSpec((2, bm, HALF), lambda i: (0, i, 0)),
        out_shape=jax.ShapeDtypeStruct((2, NP, HALF), jnp.float32),
    )(x, W1)


def _msg_body(y1_hbm, src_hbm, dst_hbm, b1_hbm,
              out_hbm, deg_hbm,
              src_v, dst_v, buf_v, zbuf_v, ebuf_v,
              ones_v, nsrc_v, ndst_v, dv_v, b1_v,
              z_s, agg_s, hs_s, hd_s,
              sem, gsem, ssem):
    c = lax.axis_index("c")
    t = lax.axis_index("s")
    base = t * SEG

    pltpu.sync_copy(src_hbm.at[t], src_v.at[pl.ds(0, EPT)])
    pltpu.sync_copy(dst_hbm.at[t], dst_v.at[pl.ds(0, EPT)])
    pltpu.sync_copy(b1_hbm.at[pl.ds(c * HALF, HALF)], b1_v)
    lane = lax.iota(jnp.int32, 16)
    for k in range((EPTP - EPT) // 16):
        pv = jnp.int32(N + (k * 16) % 240) + lane
        src_v[pl.ds(EPT + k * 16, 16)] = pv
        dst_v[pl.ds(EPT + k * 16, 16)] = pv

    for k in range(CH // 16):
        ones_v[pl.ds(k * 16, 16)] = jnp.ones((16,), jnp.float32)

    def _z1d(k, carry):
        dv_v[pl.ds(k * 16, 16)] = jnp.zeros((16,), jnp.float32)
        return carry
    lax.fori_loop(0, SEG // 16, _z1d, 0)

    pltpu.sync_copy(dv_v, hs_s.at[pl.ds(base, SEG)])
    pltpu.sync_copy(dv_v, hd_s.at[pl.ds(base, SEG)])
    plsc.subcore_barrier()

    def _hist(j, carry):
        pltpu.async_copy(ones_v, hs_s.at[src_v.at[pl.ds(j * CH, CH)]],
                         sem, add=True)
        pltpu.async_copy(ones_v, hd_s.at[dst_v.at[pl.ds(j * CH, CH)]],
                         sem, add=True)
        return carry
    lax.fori_loop(0, NCH, _hist, 0)

    def _zrow(r, carry):
        zbuf_v[r, :] = jnp.zeros((16,), jnp.float32)
        return carry
    lax.fori_loop(0, SEG, _zrow, 0)
    pltpu.sync_copy(y1_hbm.at[pl.ds(c * NP + base, SEG)], buf_v)

    def _hdrain(j, carry):
        pltpu.make_async_copy(ones_v, hs_s.at[src_v.at[pl.ds(0, CH)]],
                              sem).wait()
        pltpu.make_async_copy(ones_v, hd_s.at[dst_v.at[pl.ds(0, CH)]],
                              sem).wait()
        return carry
    lax.fori_loop(0, NCH, _hdrain, 0)
    plsc.subcore_barrier()

    def _norms(out_ref):
        def body(k, carry):
            d = jnp.maximum(dv_v[pl.ds(k * 16, 16)], jnp.float32(1.0))
            out_ref[pl.ds(k * 16, 16)] = _rsqrt_approx(d)
            return carry
        lax.fori_loop(0, SEG // 16, body, 0)

    pltpu.sync_copy(hs_s.at[pl.ds(base, SEG)], dv_v)
    _norms(nsrc_v)
    pltpu.sync_copy(hd_s.at[pl.ds(base, SEG)], dv_v)

    @pl.when(c == 0)
    def _():
        pltpu.sync_copy(dv_v, deg_hbm.at[pl.ds(base, SEG)])
    _norms(ndst_v)

    def _scale(k, carry):
        nv = nsrc_v[pl.ds(k * 16, 16)]
        for l in range(16):
            r = k * 16 + l
            buf_v[r, :] = buf_v[r, :] * nv[l]
        return carry
    lax.fori_loop(0, SEG // 16, _scale, 0)
    pltpu.sync_copy(buf_v, z_s.at[pl.ds(base, SEG)])
    pltpu.sync_copy(zbuf_v, agg_s.at[pl.ds(base, SEG)])
    plsc.subcore_barrier()

    def _g_issue(j, b):
        pltpu.async_copy(z_s.at[src_v.at[pl.ds(j * CH, CH)]], ebuf_v.at[b],
                         gsem.at[b])

    def _g_wait(b):
        pltpu.make_async_copy(z_s.at[src_v.at[pl.ds(0, CH)]], ebuf_v.at[b],
                              gsem.at[b]).wait()

    def _s_issue(j, b):
        pltpu.async_copy(ebuf_v.at[b], agg_s.at[dst_v.at[pl.ds(j * CH, CH)]],
                         ssem.at[b], add=True)

    def _s_wait(b):
        pltpu.make_async_copy(ebuf_v.at[b], agg_s.at[dst_v.at[pl.ds(0, CH)]],
                              ssem.at[b]).wait()

    def _edges():
        for b in range(NB):
            _g_issue(b, b)
        for b in range(NB - 1):
            _g_wait(b)
            _s_issue(b, b)

        def _body(o, carry):
            for b in range(NB):
                j = NB + o * NB + b
                _s_wait(b)
                _g_issue(j, b)
                b1 = (b + NB - 1) % NB
                _g_wait(b1)
                _s_issue(j - 1, b1)
            return carry
        lax.fori_loop(0, (NCH - NB) // NB, _body, 0)

        _g_wait((NCH - 1) % NB)
        _s_issue(NCH - 1, (NCH - 1) % NB)
        for b in range(NB):
            _s_wait(b)

    _edges()
    plsc.subcore_barrier()

    pltpu.sync_copy(agg_s.at[pl.ds(base, SEG)], buf_v)
    b1row = b1_v[...]

    def _mid(k, carry):
        nvd = ndst_v[pl.ds(k * 16, 16)]
        nvs = nsrc_v[pl.ds(k * 16, 16)]
        for l in range(16):
            r = k * 16 + l
            h = jnp.maximum(buf_v[r, :] * nvd[l] + b1row, jnp.float32(0.0))
            buf_v[r, :] = h * nvs[l]
        return carry
    lax.fori_loop(0, SEG // 16, _mid, 0)
    pltpu.sync_copy(buf_v, z_s.at[pl.ds(base, SEG)])
    pltpu.sync_copy(zbuf_v, agg_s.at[pl.ds(base, SEG)])
    plsc.subcore_barrier()

    _edges()
    plsc.subcore_barrier()

    pltpu.sync_copy(agg_s.at[pl.ds(base, SEG)], buf_v)
    pltpu.sync_copy(buf_v, out_hbm.at[pl.ds(c * NP + base, SEG)])


_msgpass = pl.kernel(
    _msg_body,
    out_type=(
        jax.ShapeDtypeStruct((2 * NP, HALF), jnp.float32),
        jax.ShapeDtypeStruct((NP,), jnp.float32),
    ),
    mesh=plsc.VectorSubcoreMesh(core_axis_name="c", subcore_axis_name="s"),
    compiler_params=pltpu.CompilerParams(use_tc_tiling_on_sc=False),
    scratch_types=[
        pltpu.VMEM((EPTP,), jnp.int32),
        pltpu.VMEM((EPTP,), jnp.int32),
        pltpu.VMEM((SEG, HALF), jnp.float32),
        pltpu.VMEM((SEG, HALF), jnp.float32),
        pltpu.VMEM((NB, CH, HALF), jnp.float32),
        pltpu.VMEM((CH,), jnp.float32),
        pltpu.VMEM((SEG,), jnp.float32),
        pltpu.VMEM((SEG,), jnp.float32),
        pltpu.VMEM((SEG,), jnp.float32),
        pltpu.VMEM((HALF,), jnp.float32),
        pltpu.VMEM_SHARED((NP, HALF), jnp.float32),
        pltpu.VMEM_SHARED((NP, HALF), jnp.float32),
        pltpu.VMEM_SHARED((NP,), jnp.float32),
        pltpu.VMEM_SHARED((NP,), jnp.float32),
        pltpu.SemaphoreType.DMA,
        pltpu.SemaphoreType.DMA((NB,)),
        pltpu.SemaphoreType.DMA((NB,)),
    ],
)


def _fin_body(a_ref, d_ref, w_ref, b_ref, o_ref):
    a2 = a_ref[...]
    a = jnp.concatenate([a2[0], a2[1]], axis=1)
    nd = lax.rsqrt(jnp.maximum(d_ref[...], jnp.float32(1.0)))
    o = jnp.dot(a * nd, w_ref[...], preferred_element_type=jnp.float32)
    o = o + b_ref[...]
    m = jnp.max(o, axis=1, keepdims=True)
    e = o - m
    lse = jnp.log(jnp.sum(jnp.exp(e), axis=1, keepdims=True))
    o_ref[...] = e - lse


def _final(agg, deg, W2, b2):
    bm = 1024
    return pl.pallas_call(
        _fin_body,
        grid=(NP // bm,),
        in_specs=[
            pl.BlockSpec((2, bm, HALF), lambda i: (0, i, 0)),
            pl.BlockSpec((bm, 1), lambda i: (i, 0)),
            pl.BlockSpec((H, D_OUT), lambda i: (0, 0)),
            pl.BlockSpec((1, D_OUT), lambda i: (0, 0)),
        ],
        out_specs=pl.BlockSpec((bm, D_OUT), lambda i: (i, 0)),
        out_shape=jax.ShapeDtypeStruct((N, D_OUT), jnp.float32),
    )(agg, deg, W2, b2)


@jax.jit
def kernel(x, edge_index, W1, b1, W2, b2):
    srcr = edge_index[0].reshape(NTILES, EPT)
    dstr = edge_index[1].reshape(NTILES, EPT)
    y1 = _mm1(x, W1).reshape(2 * NP, HALF)
    agg, deg = _msgpass(y1, srcr, dstr, b1)
    return _final(agg.reshape(2, NP, HALF), deg.reshape(NP, 1),
                  W2, b2.reshape(1, D_OUT))


# speedup vs baseline: 1.5891x; 1.0552x over previous
import jax
import jax.numpy as jnp
from jax import lax
from jax.experimental import pallas as pl
from jax.experimental.pallas import tpu as pltpu
from jax.experimental.pallas import tpu_sc as plsc

N = 10000
E = 320000
D_IN = 128
H = 32
D_OUT = 64

NP = 10240
NTILES = 16
SEG = NP // NTILES
CH = 512
EPT = E // NTILES
NB = 4
NCH = 40
EPTP = NCH * CH
HALF = 16
SEGR = SEG // 16


def _rsqrt_approx(d):
    i = lax.bitcast_convert_type(d, jnp.int32)
    i = jnp.int32(0x5F3759DF) - lax.shift_right_logical(i, 1)
    y = lax.bitcast_convert_type(i, jnp.float32)
    for _ in range(3):
        y = y * (jnp.float32(1.5) - jnp.float32(0.5) * d * y * y)
    return y


def _mm1_body(x_ref, w_ref, o_ref):
    x = x_ref[...]
    w = w_ref[...]
    o_ref[0] = jnp.dot(x, w[:, :HALF], preferred_element_type=jnp.float32)
    o_ref[1] = jnp.dot(x, w[:, HALF:], preferred_element_type=jnp.float32)


def _mm1(x, W1):
    bm = 1024
    return pl.pallas_call(
        _mm1_body,
        grid=(NP // bm,),
        in_specs=[
            pl.BlockSpec((bm, D_IN), lambda i: (i, 0)),
            pl.BlockSpec((D_IN, H), lambda i: (0, 0)),
        ],
        out_specs=pl.BlockSpec((2, bm, HALF), lambda i: (0, i, 0)),
        out_shape=jax.ShapeDtypeStruct((2, NP, HALF), jnp.float32),
    )(x, W1)


def _hist_body(src_hbm, dst_hbm,
               deg_hbm,
               idx_v, ones_v, dv_v,
               h_s,
               sem):
    c = lax.axis_index("c")
    t = lax.axis_index("s")
    base = t * SEG

    @pl.when(c == 0)
    def _():
        pltpu.sync_copy(src_hbm.at[t], idx_v.at[pl.ds(0, EPT)])

    @pl.when(c == 1)
    def _():
        pltpu.sync_copy(dst_hbm.at[t], idx_v.at[pl.ds(0, EPT)])
    lane = lax.iota(jnp.int32, 16)
    for k in range((EPTP - EPT) // 16):
        idx_v[pl.ds(EPT + k * 16, 16)] = jnp.int32(N + (k * 16) % 240) + lane

    for k in range(CH // 16):
        ones_v[pl.ds(k * 16, 16)] = jnp.ones((16,), jnp.float32)

    def _z1d(k, carry):
        dv_v[pl.ds(k * 16, 16)] = jnp.zeros((16,), jnp.float32)
        return carry
    lax.fori_loop(0, SEG // 16, _z1d, 0)
    pltpu.sync_copy(dv_v, h_s.at[pl.ds(base, SEG)])
    plsc.subcore_barrier()

    def _hist(j, carry):
        pltpu.async_copy(ones_v, h_s.at[idx_v.at[pl.ds(j * CH, CH)]],
                         sem, add=True)
        return carry
    lax.fori_loop(0, NCH, _hist, 0)

    def _hdrain(j, carry):
        pltpu.make_async_copy(ones_v, h_s.at[idx_v.at[pl.ds(0, CH)]],
                              sem).wait()
        return carry
    lax.fori_loop(0, NCH, _hdrain, 0)
    plsc.subcore_barrier()

    pltpu.sync_copy(h_s.at[pl.ds(base, SEG)], dv_v)
    pltpu.sync_copy(dv_v, deg_hbm.at[c, pl.ds(base, SEG)])


_hist_k = pl.kernel(
    _hist_body,
    out_type=jax.ShapeDtypeStruct((2, NP), jnp.float32),
    mesh=plsc.VectorSubcoreMesh(core_axis_name="c", subcore_axis_name="s"),
    compiler_params=pltpu.CompilerParams(use_tc_tiling_on_sc=False),
    scratch_types=[
        pltpu.VMEM((EPTP,), jnp.int32),
        pltpu.VMEM((CH,), jnp.float32),
        pltpu.VMEM((SEG,), jnp.float32),
        pltpu.VMEM_SHARED((NP,), jnp.float32),
        pltpu.SemaphoreType.DMA,
    ],
)


def _msg_body(y1_hbm, src_hbm, dst_hbm, b1_hbm, deg_hbm,
              out_hbm,
              src_v, dst_v, buf_v, zbuf_v, ebuf_v,
              nsrc_v, ndst_v, dv_v, b1_v,
              z_s, agg_s,
              sem, gsem, ssem):
    c = lax.axis_index("c")
    t = lax.axis_index("s")
    base = t * SEG

    pltpu.sync_copy(src_hbm.at[t], src_v.at[pl.ds(0, EPT)])
    pltpu.sync_copy(dst_hbm.at[t], dst_v.at[pl.ds(0, EPT)])
    pltpu.sync_copy(b1_hbm.at[pl.ds(c * HALF, HALF)], b1_v)
    pltpu.sync_copy(y1_hbm.at[pl.ds(c * NP + base, SEG)], buf_v)
    lane = lax.iota(jnp.int32, 16)
    for k in range((EPTP - EPT) // 16):
        pv = jnp.int32(N + (k * 16) % 240) + lane
        src_v[pl.ds(EPT + k * 16, 16)] = pv
        dst_v[pl.ds(EPT + k * 16, 16)] = pv

    def _zrow(r, carry):
        zbuf_v[r, :] = jnp.zeros((16,), jnp.float32)
        return carry
    lax.fori_loop(0, SEG, _zrow, 0)

    def _norms(out_ref):
        def body(k, carry):
            d = jnp.maximum(dv_v[pl.ds(k * 16, 16)], jnp.float32(1.0))
            out_ref[pl.ds(k * 16, 16)] = _rsqrt_approx(d)
            return carry
        lax.fori_loop(0, SEG // 16, body, 0)

    pltpu.sync_copy(deg_hbm.at[0, pl.ds(base, SEG)], dv_v)
    _norms(nsrc_v)
    pltpu.sync_copy(deg_hbm.at[1, pl.ds(base, SEG)], dv_v)
    _norms(ndst_v)

    def _scale(k, carry):
        nv = nsrc_v[pl.ds(k * 16, 16)]
        for l in range(16):
            r = k * 16 + l
            buf_v[r, :] = buf_v[r, :] * nv[l]
        return carry
    lax.fori_loop(0, SEG // 16, _scale, 0)
    pltpu.sync_copy(buf_v, z_s.at[pl.ds(base, SEG)])
    pltpu.sync_copy(zbuf_v, agg_s.at[pl.ds(base, SEG)])
    plsc.subcore_barrier()

    def _g_issue(j, b):
        pltpu.async_copy(z_s.at[src_v.at[pl.ds(j * CH, CH)]], ebuf_v.at[b],
                         gsem.at[b])

    def _g_wait(b):
        pltpu.make_async_copy(z_s.at[src_v.at[pl.ds(0, CH)]], ebuf_v.at[b],
                              gsem.at[b]).wait()

    def _s_issue(j, b):
        pltpu.async_copy(ebuf_v.at[b], agg_s.at[dst_v.at[pl.ds(j * CH, CH)]],
                         ssem.at[b], add=True)

    def _s_wait(b):
        pltpu.make_async_copy(ebuf_v.at[b], agg_s.at[dst_v.at[pl.ds(0, CH)]],
                              ssem.at[b]).wait()

    def _edges():
        for b in range(NB):
            _g_issue(b, b)
        for b in range(NB - 1):
            _g_wait(b)
            _s_issue(b, b)

        def _body(o, carry):
            for b in range(NB):
                j = NB + o * NB + b
                _s_wait(b)
                _g_issue(j, b)
                b1 = (b + NB - 1) % NB
                _g_wait(b1)
                _s_issue(j - 1, b1)
            return carry
        lax.fori_loop(0, (NCH - NB) // NB, _body, 0)

        _g_wait((NCH - 1) % NB)
        _s_issue(NCH - 1, (NCH - 1) % NB)
        for b in range(NB):
            _s_wait(b)

    _edges()
    plsc.subcore_barrier()

    pltpu.sync_copy(agg_s.at[pl.ds(base, SEG)], buf_v)
    b1row = b1_v[...]

    def _mid(k, carry):
        nvd = ndst_v[pl.ds(k * 16, 16)]
        nvs = nsrc_v[pl.ds(k * 16, 16)]
        for l in range(16):
            r = k * 16 + l
            h = jnp.maximum(buf_v[r, :] * nvd[l] + b1row, jnp.float32(0.0))
            buf_v[r, :] = h * nvs[l]
        return carry
    lax.fori_loop(0, SEG // 16, _mid, 0)
    pltpu.sync_copy(buf_v, z_s.at[pl.ds(base, SEG)])
    pltpu.sync_copy(zbuf_v, agg_s.at[pl.ds(base, SEG)])
    plsc.subcore_barrier()

    _edges()
    plsc.subcore_barrier()

    pltpu.sync_copy(agg_s.at[pl.ds(base, SEG)], buf_v)
    pltpu.sync_copy(buf_v, out_hbm.at[pl.ds(c * NP + base, SEG)])


_msgpass = pl.kernel(
    _msg_body,
    out_type=jax.ShapeDtypeStruct((2 * NP, HALF), jnp.float32),
    mesh=plsc.VectorSubcoreMesh(core_axis_name="c", subcore_axis_name="s"),
    compiler_params=pltpu.CompilerParams(use_tc_tiling_on_sc=False),
    scratch_types=[
        pltpu.VMEM((EPTP,), jnp.int32),
        pltpu.VMEM((EPTP,), jnp.int32),
        pltpu.VMEM((SEG, HALF), jnp.float32),
        pltpu.VMEM((SEG, HALF), jnp.float32),
        pltpu.VMEM((NB, CH, HALF), jnp.float32),
        pltpu.VMEM((SEG,), jnp.float32),
        pltpu.VMEM((SEG,), jnp.float32),
        pltpu.VMEM((SEG,), jnp.float32),
        pltpu.VMEM((HALF,), jnp.float32),
        pltpu.VMEM_SHARED((NP, HALF), jnp.float32),
        pltpu.VMEM_SHARED((NP, HALF), jnp.float32),
        pltpu.SemaphoreType.DMA,
        pltpu.SemaphoreType.DMA((NB,)),
        pltpu.SemaphoreType.DMA((NB,)),
    ],
)


def _fin_body(a_ref, d_ref, w_ref, b_ref, o_ref):
    a2 = a_ref[...]
    a = jnp.concatenate([a2[0], a2[1]], axis=1)
    nd = lax.rsqrt(jnp.maximum(d_ref[...], jnp.float32(1.0)))
    o = jnp.dot(a * nd, w_ref[...], preferred_element_type=jnp.float32)
    o = o + b_ref[...]
    m = jnp.max(o, axis=1, keepdims=True)
    e = o - m
    lse = jnp.log(jnp.sum(jnp.exp(e), axis=1, keepdims=True))
    o_ref[...] = e - lse


def _final(agg, deg, W2, b2):
    bm = 1024
    return pl.pallas_call(
        _fin_body,
        grid=(NP // bm,),
        in_specs=[
            pl.BlockSpec((2, bm, HALF), lambda i: (0, i, 0)),
            pl.BlockSpec((bm, 1), lambda i: (i, 0)),
            pl.BlockSpec((H, D_OUT), lambda i: (0, 0)),
            pl.BlockSpec((1, D_OUT), lambda i: (0, 0)),
        ],
        out_specs=pl.BlockSpec((bm, D_OUT), lambda i: (i, 0)),
        out_shape=jax.ShapeDtypeStruct((N, D_OUT), jnp.float32),
    )(agg, deg, W2, b2)


@jax.jit
def kernel(x, edge_index, W1, b1, W2, b2):
    srcr = edge_index[0].reshape(NTILES, EPT)
    dstr = edge_index[1].reshape(NTILES, EPT)
    deg = _hist_k(srcr, dstr)
    y1 = _mm1(x, W1).reshape(2 * NP, HALF)
    agg = _msgpass(y1, srcr, dstr, b1, deg)
    return _final(agg.reshape(2, NP, HALF), deg[1].reshape(NP, 1),
                  W2, b2.reshape(1, D_OUT))


# speedup vs baseline: 1.6209x; 1.0200x over previous
import jax
import jax.numpy as jnp
from jax import lax
from jax.experimental import pallas as pl
from jax.experimental.pallas import tpu as pltpu
from jax.experimental.pallas import tpu_sc as plsc

N = 10000
E = 320000
D_IN = 128
H = 32
D_OUT = 64

NP = 10240
NTILES = 16
SEG = NP // NTILES
CH = 512
EPT = E // NTILES
NB = 4
NCH = 40
EPTP = NCH * CH
HALF = 16
SEGR = SEG // 16


def _rsqrt_approx(d):
    i = lax.bitcast_convert_type(d, jnp.int32)
    i = jnp.int32(0x5F3759DF) - lax.shift_right_logical(i, 1)
    y = lax.bitcast_convert_type(i, jnp.float32)
    for _ in range(3):
        y = y * (jnp.float32(1.5) - jnp.float32(0.5) * d * y * y)
    return y


def _mm1_body(x_ref, w_ref, o_ref):
    x = x_ref[...]
    w = w_ref[...]
    o_ref[0] = jnp.dot(x, w[:, :HALF], preferred_element_type=jnp.float32)
    o_ref[1] = jnp.dot(x, w[:, HALF:], preferred_element_type=jnp.float32)


def _mm1(x, W1):
    bm = NP
    return pl.pallas_call(
        _mm1_body,
        grid=(NP // bm,),
        in_specs=[
            pl.BlockSpec((bm, D_IN), lambda i: (i, 0)),
            pl.BlockSpec((D_IN, H), lambda i: (0, 0)),
        ],
        out_specs=pl.BlockSpec((2, bm, HALF), lambda i: (0, i, 0)),
        out_shape=jax.ShapeDtypeStruct((2, NP, HALF), jnp.float32),
    )(x, W1)


def _hist_body(src_hbm, dst_hbm,
               deg_hbm,
               idx_v, ones_v, dv_v,
               h_s,
               sem):
    c = lax.axis_index("c")
    t = lax.axis_index("s")
    base = t * SEG

    @pl.when(c == 0)
    def _():
        pltpu.sync_copy(src_hbm.at[t], idx_v.at[pl.ds(0, EPT)])

    @pl.when(c == 1)
    def _():
        pltpu.sync_copy(dst_hbm.at[t], idx_v.at[pl.ds(0, EPT)])
    lane = lax.iota(jnp.int32, 16)
    for k in range((EPTP - EPT) // 16):
        idx_v[pl.ds(EPT + k * 16, 16)] = jnp.int32(N + (k * 16) % 240) + lane

    for k in range(CH // 16):
        ones_v[pl.ds(k * 16, 16)] = jnp.ones((16,), jnp.float32)

    def _z1d(k, carry):
        dv_v[pl.ds(k * 16, 16)] = jnp.zeros((16,), jnp.float32)
        return carry
    lax.fori_loop(0, SEG // 16, _z1d, 0)
    pltpu.sync_copy(dv_v, h_s.at[pl.ds(base, SEG)])
    plsc.subcore_barrier()

    def _hist(j, carry):
        pltpu.async_copy(ones_v, h_s.at[idx_v.at[pl.ds(j * CH, CH)]],
                         sem, add=True)
        return carry
    lax.fori_loop(0, NCH, _hist, 0)

    def _hdrain(j, carry):
        pltpu.make_async_copy(ones_v, h_s.at[idx_v.at[pl.ds(0, CH)]],
                              sem).wait()
        return carry
    lax.fori_loop(0, NCH, _hdrain, 0)
    plsc.subcore_barrier()

    pltpu.sync_copy(h_s.at[pl.ds(base, SEG)], dv_v)
    pltpu.sync_copy(dv_v, deg_hbm.at[c, pl.ds(base, SEG)])


_hist_k = pl.kernel(
    _hist_body,
    out_type=jax.ShapeDtypeStruct((2, NP), jnp.float32),
    mesh=plsc.VectorSubcoreMesh(core_axis_name="c", subcore_axis_name="s"),
    compiler_params=pltpu.CompilerParams(use_tc_tiling_on_sc=False),
    scratch_types=[
        pltpu.VMEM((EPTP,), jnp.int32),
        pltpu.VMEM((CH,), jnp.float32),
        pltpu.VMEM((SEG,), jnp.float32),
        pltpu.VMEM_SHARED((NP,), jnp.float32),
        pltpu.SemaphoreType.DMA,
    ],
)


def _msg_body(y1_hbm, src_hbm, dst_hbm, b1_hbm, deg_hbm,
              out_hbm,
              src_v, dst_v, buf_v, zbuf_v, ebuf_v,
              nsrc_v, ndst_v, dv_v, b1_v,
              z_s, agg_s,
              sem, gsem, ssem):
    c = lax.axis_index("c")
    t = lax.axis_index("s")
    base = t * SEG

    pltpu.sync_copy(src_hbm.at[t], src_v.at[pl.ds(0, EPT)])
    pltpu.sync_copy(dst_hbm.at[t], dst_v.at[pl.ds(0, EPT)])
    pltpu.sync_copy(b1_hbm.at[pl.ds(c * HALF, HALF)], b1_v)
    pltpu.sync_copy(y1_hbm.at[pl.ds(c * NP + base, SEG)], buf_v)
    lane = lax.iota(jnp.int32, 16)
    for k in range((EPTP - EPT) // 16):
        pv = jnp.int32(N + (k * 16) % 240) + lane
        src_v[pl.ds(EPT + k * 16, 16)] = pv
        dst_v[pl.ds(EPT + k * 16, 16)] = pv

    def _zrow(r, carry):
        zbuf_v[r, :] = jnp.zeros((16,), jnp.float32)
        return carry
    lax.fori_loop(0, SEG, _zrow, 0)

    def _norms(out_ref):
        def body(k, carry):
            d = jnp.maximum(dv_v[pl.ds(k * 16, 16)], jnp.float32(1.0))
            out_ref[pl.ds(k * 16, 16)] = _rsqrt_approx(d)
            return carry
        lax.fori_loop(0, SEG // 16, body, 0)

    pltpu.sync_copy(deg_hbm.at[0, pl.ds(base, SEG)], dv_v)
    _norms(nsrc_v)
    pltpu.sync_copy(deg_hbm.at[1, pl.ds(base, SEG)], dv_v)
    _norms(ndst_v)

    def _scale(k, carry):
        nv = nsrc_v[pl.ds(k * 16, 16)]
        for l in range(16):
            r = k * 16 + l
            buf_v[r, :] = buf_v[r, :] * nv[l]
        return carry
    lax.fori_loop(0, SEG // 16, _scale, 0)
    pltpu.sync_copy(buf_v, z_s.at[pl.ds(base, SEG)])
    pltpu.sync_copy(zbuf_v, agg_s.at[pl.ds(base, SEG)])
    plsc.subcore_barrier()

    def _g_issue(j, b):
        pltpu.async_copy(z_s.at[src_v.at[pl.ds(j * CH, CH)]], ebuf_v.at[b],
                         gsem.at[b])

    def _g_wait(b):
        pltpu.make_async_copy(z_s.at[src_v.at[pl.ds(0, CH)]], ebuf_v.at[b],
                              gsem.at[b]).wait()

    def _s_issue(j, b):
        pltpu.async_copy(ebuf_v.at[b], agg_s.at[dst_v.at[pl.ds(j * CH, CH)]],
                         ssem.at[b], add=True)

    def _s_wait(b):
        pltpu.make_async_copy(ebuf_v.at[b], agg_s.at[dst_v.at[pl.ds(0, CH)]],
                              ssem.at[b]).wait()

    def _edges():
        for b in range(NB):
            _g_issue(b, b)
        for b in range(NB - 1):
            _g_wait(b)
            _s_issue(b, b)

        def _body(o, carry):
            for b in range(NB):
                j = NB + o * NB + b
                _s_wait(b)
                _g_issue(j, b)
                b1 = (b + NB - 1) % NB
                _g_wait(b1)
                _s_issue(j - 1, b1)
            return carry
        lax.fori_loop(0, (NCH - NB) // NB, _body, 0)

        _g_wait((NCH - 1) % NB)
        _s_issue(NCH - 1, (NCH - 1) % NB)
        for b in range(NB):
            _s_wait(b)

    _edges()
    plsc.subcore_barrier()

    pltpu.sync_copy(agg_s.at[pl.ds(base, SEG)], buf_v)
    b1row = b1_v[...]

    def _mid(k, carry):
        nvd = ndst_v[pl.ds(k * 16, 16)]
        nvs = nsrc_v[pl.ds(k * 16, 16)]
        for l in range(16):
            r = k * 16 + l
            h = jnp.maximum(buf_v[r, :] * nvd[l] + b1row, jnp.float32(0.0))
            buf_v[r, :] = h * nvs[l]
        return carry
    lax.fori_loop(0, SEG // 16, _mid, 0)
    pltpu.sync_copy(buf_v, z_s.at[pl.ds(base, SEG)])
    pltpu.sync_copy(zbuf_v, agg_s.at[pl.ds(base, SEG)])
    plsc.subcore_barrier()

    _edges()
    plsc.subcore_barrier()

    pltpu.sync_copy(agg_s.at[pl.ds(base, SEG)], buf_v)
    pltpu.sync_copy(buf_v, out_hbm.at[pl.ds(c * NP + base, SEG)])


_msgpass = pl.kernel(
    _msg_body,
    out_type=jax.ShapeDtypeStruct((2 * NP, HALF), jnp.float32),
    mesh=plsc.VectorSubcoreMesh(core_axis_name="c", subcore_axis_name="s"),
    compiler_params=pltpu.CompilerParams(use_tc_tiling_on_sc=False),
    scratch_types=[
        pltpu.VMEM((EPTP,), jnp.int32),
        pltpu.VMEM((EPTP,), jnp.int32),
        pltpu.VMEM((SEG, HALF), jnp.float32),
        pltpu.VMEM((SEG, HALF), jnp.float32),
        pltpu.VMEM((NB, CH, HALF), jnp.float32),
        pltpu.VMEM((SEG,), jnp.float32),
        pltpu.VMEM((SEG,), jnp.float32),
        pltpu.VMEM((SEG,), jnp.float32),
        pltpu.VMEM((HALF,), jnp.float32),
        pltpu.VMEM_SHARED((NP, HALF), jnp.float32),
        pltpu.VMEM_SHARED((NP, HALF), jnp.float32),
        pltpu.SemaphoreType.DMA,
        pltpu.SemaphoreType.DMA((NB,)),
        pltpu.SemaphoreType.DMA((NB,)),
    ],
)


def _fin_body(a_ref, d_ref, w_ref, b_ref, o_ref):
    a2 = a_ref[...]
    a = jnp.concatenate([a2[0], a2[1]], axis=1)
    nd = lax.rsqrt(jnp.maximum(d_ref[...], jnp.float32(1.0)))
    o = jnp.dot(a * nd, w_ref[...], preferred_element_type=jnp.float32)
    o = o + b_ref[...]
    m = jnp.max(o, axis=1, keepdims=True)
    e = o - m
    lse = jnp.log(jnp.sum(jnp.exp(e), axis=1, keepdims=True))
    o_ref[...] = e - lse


def _final(agg, deg, W2, b2):
    bm = NP
    return pl.pallas_call(
        _fin_body,
        grid=(NP // bm,),
        in_specs=[
            pl.BlockSpec((2, bm, HALF), lambda i: (0, i, 0)),
            pl.BlockSpec((bm, 1), lambda i: (i, 0)),
            pl.BlockSpec((H, D_OUT), lambda i: (0, 0)),
            pl.BlockSpec((1, D_OUT), lambda i: (0, 0)),
        ],
        out_specs=pl.BlockSpec((bm, D_OUT), lambda i: (i, 0)),
        out_shape=jax.ShapeDtypeStruct((N, D_OUT), jnp.float32),
    )(agg, deg, W2, b2)


@jax.jit
def kernel(x, edge_index, W1, b1, W2, b2):
    srcr = edge_index[0].reshape(NTILES, EPT)
    dstr = edge_index[1].reshape(NTILES, EPT)
    deg = _hist_k(srcr, dstr)
    y1 = _mm1(x, W1).reshape(2 * NP, HALF)
    agg = _msgpass(y1, srcr, dstr, b1, deg)
    return _final(agg.reshape(2, NP, HALF), deg[1].reshape(NP, 1),
                  W2, b2.reshape(1, D_OUT))


# speedup vs baseline: 1.6220x; 1.0007x over previous
import jax
import jax.numpy as jnp
from jax import lax
from jax.experimental import pallas as pl
from jax.experimental.pallas import tpu as pltpu
from jax.experimental.pallas import tpu_sc as plsc

N = 10000
E = 320000
D_IN = 128
H = 32
D_OUT = 64

NP = 10240
NTILES = 16
SEG = NP // NTILES
CH = 512
EPT = E // NTILES
NB = 4
NCH = 40
EPTP = NCH * CH
HALF = 16


def _rsqrt_approx(d):
    i = lax.bitcast_convert_type(d, jnp.int32)
    i = jnp.int32(0x5F3759DF) - lax.shift_right_logical(i, 1)
    y = lax.bitcast_convert_type(i, jnp.float32)
    for _ in range(3):
        y = y * (jnp.float32(1.5) - jnp.float32(0.5) * d * y * y)
    return y


def _mm1_body(x_ref, w_ref, o_ref):
    x = x_ref[...]
    w = w_ref[...]
    o_ref[0] = jnp.dot(x, w[:, :HALF], preferred_element_type=jnp.float32)
    o_ref[1] = jnp.dot(x, w[:, HALF:], preferred_element_type=jnp.float32)


def _mm1(x, W1):
    bm = NP
    return pl.pallas_call(
        _mm1_body,
        grid=(NP // bm,),
        in_specs=[
            pl.BlockSpec((bm, D_IN), lambda i: (i, 0)),
            pl.BlockSpec((D_IN, H), lambda i: (0, 0)),
        ],
        out_specs=pl.BlockSpec((2, bm, HALF), lambda i: (0, i, 0)),
        out_shape=jax.ShapeDtypeStruct((2, NP, HALF), jnp.float32),
    )(x, W1)


def _hist_body(src_hbm, dst_hbm,
               deg_hbm,
               idx_v, ones_v, dv_v,
               h_s,
               sem):
    c = lax.axis_index("c")
    t = lax.axis_index("s")
    base = t * SEG

    @pl.when(c == 0)
    def _():
        pltpu.sync_copy(src_hbm.at[t], idx_v.at[pl.ds(0, EPT)])

    @pl.when(c == 1)
    def _():
        pltpu.sync_copy(dst_hbm.at[t], idx_v.at[pl.ds(0, EPT)])
    lane = lax.iota(jnp.int32, 16)
    for k in range((EPTP - EPT) // 16):
        idx_v[pl.ds(EPT + k * 16, 16)] = jnp.int32(N + (k * 16) % 240) + lane

    for k in range(CH // 16):
        ones_v[pl.ds(k * 16, 16)] = jnp.ones((16,), jnp.float32)

    def _z1d(k, carry):
        dv_v[pl.ds(k * 16, 16)] = jnp.zeros((16,), jnp.float32)
        return carry
    lax.fori_loop(0, SEG // 16, _z1d, 0)
    pltpu.sync_copy(dv_v, h_s.at[pl.ds(base, SEG)])
    plsc.subcore_barrier()

    def _hist(j, carry):
        pltpu.async_copy(ones_v, h_s.at[idx_v.at[pl.ds(j * CH, CH)]],
                         sem, add=True)
        return carry
    lax.fori_loop(0, NCH, _hist, 0)

    def _hdrain(j, carry):
        pltpu.make_async_copy(ones_v, h_s.at[idx_v.at[pl.ds(0, CH)]],
                              sem).wait()
        return carry
    lax.fori_loop(0, NCH, _hdrain, 0)
    plsc.subcore_barrier()

    pltpu.sync_copy(h_s.at[pl.ds(base, SEG)], dv_v)
    pltpu.sync_copy(dv_v, deg_hbm.at[c, pl.ds(base, SEG)])


_hist_k = pl.kernel(
    _hist_body,
    out_type=jax.ShapeDtypeStruct((2, NP), jnp.float32),
    mesh=plsc.VectorSubcoreMesh(core_axis_name="c", subcore_axis_name="s"),
    compiler_params=pltpu.CompilerParams(use_tc_tiling_on_sc=False),
    scratch_types=[
        pltpu.VMEM((EPTP,), jnp.int32),
        pltpu.VMEM((CH,), jnp.float32),
        pltpu.VMEM((SEG,), jnp.float32),
        pltpu.VMEM_SHARED((NP,), jnp.float32),
        pltpu.SemaphoreType.DMA,
    ],
)


def _msg_body(y1_hbm, src_hbm, dst_hbm, b1_hbm, deg_hbm,
              out_hbm,
              src_v, dst_v, buf_v, zbuf_v, ebuf_v,
              nsrc_v, ndst_v, dv_v, b1_v,
              z_s, agg_s,
              sem, gsem, ssem):
    c = lax.axis_index("c")
    t = lax.axis_index("s")
    base = t * SEG

    pltpu.sync_copy(src_hbm.at[t], src_v.at[pl.ds(0, EPT)])
    pltpu.sync_copy(dst_hbm.at[t], dst_v.at[pl.ds(0, EPT)])
    pltpu.sync_copy(b1_hbm.at[pl.ds(c * HALF, HALF)], b1_v)
    pltpu.sync_copy(y1_hbm.at[pl.ds(c * NP + base, SEG)], buf_v)
    lane = lax.iota(jnp.int32, 16)
    for k in range((EPTP - EPT) // 16):
        pv = jnp.int32(N + (k * 16) % 240) + lane
        src_v[pl.ds(EPT + k * 16, 16)] = pv
        dst_v[pl.ds(EPT + k * 16, 16)] = pv

    def _zrow(r, carry):
        zbuf_v[r, :] = jnp.zeros((16,), jnp.float32)
        return carry
    lax.fori_loop(0, SEG, _zrow, 0)

    def _norms(out_ref):
        def body(k, carry):
            d = jnp.maximum(dv_v[pl.ds(k * 16, 16)], jnp.float32(1.0))
            out_ref[pl.ds(k * 16, 16)] = _rsqrt_approx(d)
            return carry
        lax.fori_loop(0, SEG // 16, body, 0)

    pltpu.sync_copy(deg_hbm.at[0, pl.ds(base, SEG)], dv_v)
    _norms(nsrc_v)
    pltpu.sync_copy(deg_hbm.at[1, pl.ds(base, SEG)], dv_v)
    _norms(ndst_v)

    def _scale(k, carry):
        nv = nsrc_v[pl.ds(k * 16, 16)]
        for l in range(16):
            r = k * 16 + l
            buf_v[r, :] = buf_v[r, :] * nv[l]
        return carry
    lax.fori_loop(0, SEG // 16, _scale, 0)
    pltpu.sync_copy(buf_v, z_s.at[pl.ds(base, SEG)])
    pltpu.sync_copy(zbuf_v, agg_s.at[pl.ds(base, SEG)])
    plsc.subcore_barrier()

    def _g_issue(j, b):
        pltpu.async_copy(z_s.at[src_v.at[pl.ds(j * CH, CH)]], ebuf_v.at[b],
                         gsem.at[b])

    def _g_wait(b):
        pltpu.make_async_copy(z_s.at[src_v.at[pl.ds(0, CH)]], ebuf_v.at[b],
                              gsem.at[b]).wait()

    def _s_issue(j, b):
        pltpu.async_copy(ebuf_v.at[b], agg_s.at[dst_v.at[pl.ds(j * CH, CH)]],
                         ssem.at[b], add=True)

    def _s_wait(b):
        pltpu.make_async_copy(ebuf_v.at[b], agg_s.at[dst_v.at[pl.ds(0, CH)]],
                              ssem.at[b]).wait()

    def _edges():
        for b in range(NB):
            _g_issue(b, b)
        for b in range(NB - 1):
            _g_wait(b)
            _s_issue(b, b)

        def _body(o, carry):
            for b in range(NB):
                j = NB + o * NB + b
                _s_wait(b)
                _g_issue(j, b)
                b1 = (b + NB - 1) % NB
                _g_wait(b1)
                _s_issue(j - 1, b1)
            return carry
        lax.fori_loop(0, (NCH - NB) // NB, _body, 0)

        _g_wait((NCH - 1) % NB)
        _s_issue(NCH - 1, (NCH - 1) % NB)
        for b in range(NB):
            _s_wait(b)

    _edges()
    plsc.subcore_barrier()

    pltpu.sync_copy(agg_s.at[pl.ds(base, SEG)], buf_v)
    b1row = b1_v[...]

    def _mid(k, carry):
        nvd = ndst_v[pl.ds(k * 16, 16)]
        nvs = nsrc_v[pl.ds(k * 16, 16)]
        for l in range(16):
            r = k * 16 + l
            h = jnp.maximum(buf_v[r, :] * nvd[l] + b1row, jnp.float32(0.0))
            buf_v[r, :] = h * nvs[l]
        return carry
    lax.fori_loop(0, SEG // 16, _mid, 0)
    pltpu.sync_copy(buf_v, z_s.at[pl.ds(base, SEG)])
    pltpu.sync_copy(zbuf_v, agg_s.at[pl.ds(base, SEG)])
    plsc.subcore_barrier()

    _edges()
    plsc.subcore_barrier()

    pltpu.sync_copy(agg_s.at[pl.ds(base, SEG)], buf_v)
    pltpu.sync_copy(buf_v, out_hbm.at[pl.ds(c * NP + base, SEG)])


_msgpass = pl.kernel(
    _msg_body,
    out_type=jax.ShapeDtypeStruct((2 * NP, HALF), jnp.float32),
    mesh=plsc.VectorSubcoreMesh(core_axis_name="c", subcore_axis_name="s"),
    compiler_params=pltpu.CompilerParams(use_tc_tiling_on_sc=False),
    scratch_types=[
        pltpu.VMEM((EPTP,), jnp.int32),
        pltpu.VMEM((EPTP,), jnp.int32),
        pltpu.VMEM((SEG, HALF), jnp.float32),
        pltpu.VMEM((SEG, HALF), jnp.float32),
        pltpu.VMEM((NB, CH, HALF), jnp.float32),
        pltpu.VMEM((SEG,), jnp.float32),
        pltpu.VMEM((SEG,), jnp.float32),
        pltpu.VMEM((SEG,), jnp.float32),
        pltpu.VMEM((HALF,), jnp.float32),
        pltpu.VMEM_SHARED((NP, HALF), jnp.float32),
        pltpu.VMEM_SHARED((NP, HALF), jnp.float32),
        pltpu.SemaphoreType.DMA,
        pltpu.SemaphoreType.DMA((NB,)),
        pltpu.SemaphoreType.DMA((NB,)),
    ],
)


def _fin_body(a_ref, d_ref, w_ref, b_ref, o_ref):
    a2 = a_ref[...]
    a = jnp.concatenate([a2[0], a2[1]], axis=1)
    nd = lax.rsqrt(jnp.maximum(d_ref[...], jnp.float32(1.0)))
    o = jnp.dot(a * nd, w_ref[...], preferred_element_type=jnp.float32)
    o = o + b_ref[...]
    m = jnp.max(o, axis=1, keepdims=True)
    e = o - m
    lse = jnp.log(jnp.sum(jnp.exp(e), axis=1, keepdims=True))
    o_ref[...] = e - lse


def _final(agg, deg, W2, b2):
    bm = NP
    return pl.pallas_call(
        _fin_body,
        grid=(NP // bm,),
        in_specs=[
            pl.BlockSpec((2, bm, HALF), lambda i: (0, i, 0)),
            pl.BlockSpec((bm, 1), lambda i: (i, 0)),
            pl.BlockSpec((H, D_OUT), lambda i: (0, 0)),
            pl.BlockSpec((1, D_OUT), lambda i: (0, 0)),
        ],
        out_specs=pl.BlockSpec((bm, D_OUT), lambda i: (i, 0)),
        out_shape=jax.ShapeDtypeStruct((N, D_OUT), jnp.float32),
    )(agg, deg, W2, b2)


@jax.jit
def kernel(x, edge_index, W1, b1, W2, b2):
    srcr = edge_index[0].reshape(NTILES, EPT)
    dstr = edge_index[1].reshape(NTILES, EPT)
    deg = _hist_k(srcr, dstr)
    y1 = _mm1(x, W1).reshape(2 * NP, HALF)
    agg = _msgpass(y1, srcr, dstr, b1, deg)
    return _final(agg.reshape(2, NP, HALF), deg[1].reshape(NP, 1),
                  W2, b2.reshape(1, D_OUT))
